# Initial kernel scaffold; baseline (speedup 1.0000x reference)
import jax
import jax.numpy as jnp
from jax.experimental import pallas as pl


def kernel(x, edge_index, edge_weight, W1, b1, W2, b2):
    raise NotImplementedError("write your pallas kernel here")


# speedup vs baseline: 8.7176x; 8.7176x over previous
import functools

import jax
import jax.numpy as jnp
from jax import lax
from jax.experimental import pallas as pl
from jax.experimental.pallas import tpu as pltpu
from jax.experimental.pallas import tpu_sc as plsc

N = 10000
D = 128
E = 320000

NC = 2
NS = 16
NW = NC * NS
CH = 128
EPW = -(-E // (NW * CH)) * CH
E_PAD = EPW * NW
NCHUNK = EPW // CH
N_PAD = NS * (-(-N // (NS * 8)) * 8)

_mesh = plsc.VectorSubcoreMesh(core_axis_name="c", subcore_axis_name="s")


@functools.partial(
    pl.kernel,
    out_type=jax.ShapeDtypeStruct((NC, 1, N_PAD), jnp.float32),
    mesh=_mesh,
    scratch_types=[
        pltpu.VMEM((CH,), jnp.int32),
        pltpu.VMEM((CH,), jnp.float32),
        pltpu.VMEM((N_PAD,), jnp.float32),
        pltpu.VMEM_SHARED((N_PAD,), jnp.float32),
    ],
)
def _deg_kernel(dst_hbm, ew_hbm, out_hbm, dst_v, ew_v, zer_v, deg_sh):
    cid = lax.axis_index("c")
    sid = lax.axis_index("s")
    wid = cid * NS + sid

    @pl.when(sid == 0)
    def _():
        def zfill(i, _):
            zer_v[pl.ds(i * 16, 16)] = jnp.zeros((16,), jnp.float32)
            return 0
        lax.fori_loop(0, N_PAD // 16, zfill, 0)
        pltpu.sync_copy(zer_v, deg_sh)
    plsc.subcore_barrier()

    def body(c, _):
        base = wid * EPW + c * CH
        pltpu.sync_copy(dst_hbm.at[pl.ds(base, CH)], dst_v)
        pltpu.sync_copy(ew_hbm.at[pl.ds(base, CH)], ew_v)
        pltpu.sync_copy(ew_v, deg_sh.at[dst_v], add=True)
        return 0
    lax.fori_loop(0, NCHUNK, body, 0)
    plsc.subcore_barrier()

    @pl.when(sid == 0)
    def _():
        pltpu.sync_copy(deg_sh, out_hbm.at[cid, 0])


@functools.partial(
    pl.kernel,
    out_type=jax.ShapeDtypeStruct((NC, N_PAD, D), jnp.float32),
    mesh=_mesh,
    scratch_types=[
        pltpu.VMEM((CH, D), jnp.float32),
        pltpu.VMEM((CH,), jnp.int32),
        pltpu.VMEM((CH,), jnp.int32),
        pltpu.VMEM((CH,), jnp.float32),
        pltpu.VMEM_SHARED((N_PAD, D), jnp.float32),
        pltpu.SemaphoreType.DMA,
    ],
)
def _edge_kernel(g_hbm, src_hbm, dst_hbm, ew_hbm, out_hbm,
                 rows_v, src_v, dst_v, ew_v, acc_sh, sem):
    cid = lax.axis_index("c")
    sid = lax.axis_index("s")
    wid = cid * NS + sid

    def zfill(i, _):
        for j in range(D // 16):
            rows_v[i, pl.ds(j * 16, 16)] = jnp.zeros((16,), jnp.float32)
        return 0
    lax.fori_loop(0, CH, zfill, 0)
    per = N_PAD // NS
    for k in range(4):
        pltpu.sync_copy(rows_v,
                        acc_sh.at[pl.ds(sid * per + k * CH, CH)])
    pltpu.sync_copy(rows_v.at[pl.ds(0, per - 4 * CH)],
                    acc_sh.at[pl.ds(sid * per + 4 * CH, per - 4 * CH)])
    plsc.subcore_barrier()

    def body(c, _):
        base = wid * EPW + c * CH
        pltpu.sync_copy(src_hbm.at[pl.ds(base, CH)], src_v)
        pltpu.sync_copy(dst_hbm.at[pl.ds(base, CH)], dst_v)
        pltpu.sync_copy(ew_hbm.at[pl.ds(base, CH)], ew_v)
        pltpu.async_copy(g_hbm.at[src_v], rows_v, sem).wait()

        def scale(i16, _):
            wv = ew_v[pl.ds(i16 * 16, 16)]
            for l in range(16):
                w = wv[l]
                r = i16 * 16 + l
                for j in range(D // 16):
                    rows_v[r, pl.ds(j * 16, 16)] = (
                        rows_v[r, pl.ds(j * 16, 16)] * w)
            return 0
        lax.fori_loop(0, CH // 16, scale, 0)
        pltpu.sync_copy(rows_v, acc_sh.at[dst_v], add=True)
        return 0
    lax.fori_loop(0, NCHUNK, body, 0)
    plsc.subcore_barrier()

    per_out = N_PAD // NS
    pltpu.sync_copy(acc_sh.at[pl.ds(sid * per_out, per_out)],
                    out_hbm.at[cid, pl.ds(sid * per_out, per_out)])


_BLK = 2000


def _tc_head(deg_ref, x_ref, w_ref, dis_ref, g_ref):
    deg = deg_ref[:, 0:1] + deg_ref[:, 1:2] + 1.0
    dis = lax.rsqrt(deg)
    dis_ref[...] = dis
    g_ref[...] = jnp.dot(x_ref[...], w_ref[...],
                         preferred_element_type=jnp.float32) * dis


def _tc_mid(acc_ref, g_ref, dis_ref, b_ref, w_ref, g2_ref):
    dis = dis_ref[...]
    t = dis * (acc_ref[0] + acc_ref[1] + g_ref[...]) + b_ref[...]
    h = jnp.maximum(t, 0.0)
    g2_ref[...] = jnp.dot(h, w_ref[...],
                          preferred_element_type=jnp.float32) * dis


def _tc_tail(acc_ref, g_ref, dis_ref, b_ref, out_ref):
    out_ref[...] = (dis_ref[...] * (acc_ref[0] + acc_ref[1] + g_ref[...])
                    + b_ref[...])


def _row_spec(block):
    return pl.BlockSpec((block, D), lambda i: (i, 0))


_tc_head_call = pl.pallas_call(
    _tc_head,
    grid=(N // _BLK,),
    in_specs=[
        pl.BlockSpec((_BLK, 2), lambda i: (i, 0)),
        _row_spec(_BLK),
        pl.BlockSpec((D, D), lambda i: (0, 0)),
    ],
    out_specs=[
        pl.BlockSpec((_BLK, 1), lambda i: (i, 0)),
        _row_spec(_BLK),
    ],
    out_shape=[
        jax.ShapeDtypeStruct((N, 1), jnp.float32),
        jax.ShapeDtypeStruct((N, D), jnp.float32),
    ],
)

_tc_mid_call = pl.pallas_call(
    _tc_mid,
    grid=(N // _BLK,),
    in_specs=[
        pl.BlockSpec((NC, _BLK, D), lambda i: (0, i, 0)),
        _row_spec(_BLK),
        pl.BlockSpec((_BLK, 1), lambda i: (i, 0)),
        pl.BlockSpec((1, D), lambda i: (0, 0)),
        pl.BlockSpec((D, D), lambda i: (0, 0)),
    ],
    out_specs=_row_spec(_BLK),
    out_shape=jax.ShapeDtypeStruct((N, D), jnp.float32),
)

_tc_tail_call = pl.pallas_call(
    _tc_tail,
    grid=(N // _BLK,),
    in_specs=[
        pl.BlockSpec((NC, _BLK, D), lambda i: (0, i, 0)),
        _row_spec(_BLK),
        pl.BlockSpec((_BLK, 1), lambda i: (i, 0)),
        pl.BlockSpec((1, D), lambda i: (0, 0)),
    ],
    out_specs=_row_spec(_BLK),
    out_shape=jax.ShapeDtypeStruct((N, D), jnp.float32),
)


def kernel(x, edge_index, edge_weight, W1, b1, W2, b2):
    src = edge_index[0].astype(jnp.int32)
    dst = edge_index[1].astype(jnp.int32)
    ew = edge_weight.astype(jnp.float32)
    pad = E_PAD - E
    src = jnp.concatenate([src, jnp.zeros((pad,), jnp.int32)])
    dst = jnp.concatenate([dst, jnp.zeros((pad,), jnp.int32)])
    ew = jnp.concatenate([ew, jnp.zeros((pad,), jnp.float32)])

    deg_p = _deg_kernel(dst, ew)
    deg_t = jnp.transpose(deg_p.reshape(NC, N_PAD))
    dis, g1 = _tc_head_call(deg_t, x, W1)
    acc1 = _edge_kernel(g1, src, dst, ew)
    g2 = _tc_mid_call(acc1, g1, dis, b1.reshape(1, D), W2)
    acc2 = _edge_kernel(g2, src, dst, ew)
    out = _tc_tail_call(acc2, g2, dis, b2.reshape(1, D))
    return out


# speedup vs baseline: 8.8488x; 1.0150x over previous
import functools

import jax
import jax.numpy as jnp
from jax import lax
from jax.experimental import pallas as pl
from jax.experimental.pallas import tpu as pltpu
from jax.experimental.pallas import tpu_sc as plsc

N = 10000
D = 128
E = 320000

NC = 2
NS = 16
NW = NC * NS
CH = 128
NCHUNK = 80
PH = 16
NPHASE = NCHUNK // PH
EPW = NCHUNK * CH
E_PAD = EPW * NW
N_PAD = NS * (-(-N // (NS * 8)) * 8)

_mesh = plsc.VectorSubcoreMesh(core_axis_name="c", subcore_axis_name="s")


@functools.partial(
    pl.kernel,
    out_type=jax.ShapeDtypeStruct((NC, 1, N_PAD), jnp.float32),
    mesh=_mesh,
    scratch_types=[
        pltpu.VMEM((NCHUNK, CH), jnp.int32),
        pltpu.VMEM((NCHUNK, CH), jnp.float32),
        pltpu.VMEM((N_PAD,), jnp.float32),
        pltpu.VMEM_SHARED((N_PAD,), jnp.float32),
        pltpu.SemaphoreType.DMA,
    ],
)
def _deg_kernel(dst_hbm, ew_hbm, out_hbm, dst_v, ew_v, zer_v, deg_sh, sem):
    cid = lax.axis_index("c")
    sid = lax.axis_index("s")
    wid = cid * NS + sid

    pltpu.async_copy(dst_hbm.at[wid], dst_v, sem)
    pltpu.async_copy(ew_hbm.at[wid], ew_v, sem)

    @pl.when(sid == 0)
    def _():
        def zfill(i, _):
            zer_v[pl.ds(i * 16, 16)] = jnp.zeros((16,), jnp.float32)
            return 0
        lax.fori_loop(0, N_PAD // 16, zfill, 0)
        pltpu.sync_copy(zer_v, deg_sh)

    pltpu.make_async_copy(dst_hbm.at[wid], dst_v, sem).wait()
    pltpu.make_async_copy(ew_hbm.at[wid], ew_v, sem).wait()
    plsc.subcore_barrier()

    def body(g, _):
        for j in range(8):
            c = g * 8 + j
            pltpu.async_copy(ew_v.at[c], deg_sh.at[dst_v.at[c]], sem,
                             add=True)
        for j in range(8):
            pltpu.make_async_copy(ew_v.at[0], deg_sh.at[dst_v.at[0]],
                                  sem).wait()
        return 0
    lax.fori_loop(0, NCHUNK // 8, body, 0)
    plsc.subcore_barrier()

    @pl.when(sid == 0)
    def _():
        pltpu.sync_copy(deg_sh, out_hbm.at[cid, 0])


@functools.partial(
    pl.kernel,
    out_type=jax.ShapeDtypeStruct((NC, N_PAD, D), jnp.float32),
    mesh=_mesh,
    scratch_types=[
        pltpu.VMEM((PH, CH), jnp.int32),
        pltpu.VMEM((PH, CH), jnp.int32),
        pltpu.VMEM((PH, CH), jnp.float32),
        pltpu.VMEM((CH, D), jnp.float32),
        pltpu.VMEM((CH, D), jnp.float32),
        pltpu.VMEM_SHARED((N_PAD, D), jnp.float32),
        pltpu.SemaphoreType.DMA,
        pltpu.SemaphoreType.DMA,
        pltpu.SemaphoreType.DMA,
        pltpu.SemaphoreType.DMA,
        pltpu.SemaphoreType.DMA,
    ],
)
def _edge_kernel(g_hbm, src_hbm, dst_hbm, ew_hbm, out_hbm,
                 src_v, dst_v, ew_v, rows_a, rows_b, acc_sh,
                 sga, sgb, ssa, ssb, sidx):
    cid = lax.axis_index("c")
    sid = lax.axis_index("s")
    wid = cid * NS + sid

    def zfill(i, _):
        for j in range(D // 16):
            rows_a[i, pl.ds(j * 16, 16)] = jnp.zeros((16,), jnp.float32)
        return 0
    lax.fori_loop(0, CH, zfill, 0)
    per = N_PAD // NS
    for k in range(4):
        pltpu.sync_copy(rows_a, acc_sh.at[pl.ds(sid * per + k * CH, CH)])
    pltpu.sync_copy(rows_a.at[pl.ds(0, per - 4 * CH)],
                    acc_sh.at[pl.ds(sid * per + 4 * CH, per - 4 * CH)])
    plsc.subcore_barrier()

    def gather(c, buf, sem):
        pltpu.async_copy(g_hbm.at[src_v.at[c]], buf, sem)

    def gather_wait(buf, sem):
        pltpu.make_async_copy(g_hbm.at[src_v.at[0]], buf, sem).wait()

    def scatter(c, buf, sem):
        pltpu.async_copy(buf, acc_sh.at[dst_v.at[c]], sem, add=True)

    def scatter_wait(buf, sem):
        pltpu.make_async_copy(buf, acc_sh.at[dst_v.at[0]], sem).wait()

    def scale(c, buf):
        def body16(i16, _):
            wv = ew_v[c, pl.ds(i16 * 16, 16)]
            for l in range(16):
                w = wv[l]
                r = i16 * 16 + l
                for j in range(D // 16):
                    buf[r, pl.ds(j * 16, 16)] = buf[r, pl.ds(j * 16, 16)] * w
            return 0
        lax.fori_loop(0, CH // 16, body16, 0)

    def phase(p, _):
        pltpu.async_copy(src_hbm.at[wid, pl.ds(p * PH, PH)], src_v, sidx)
        pltpu.async_copy(dst_hbm.at[wid, pl.ds(p * PH, PH)], dst_v, sidx)
        pltpu.async_copy(ew_hbm.at[wid, pl.ds(p * PH, PH)], ew_v, sidx)
        pltpu.make_async_copy(src_hbm.at[wid, pl.ds(0, PH)], src_v,
                              sidx).wait()
        pltpu.make_async_copy(dst_hbm.at[wid, pl.ds(0, PH)], dst_v,
                              sidx).wait()
        pltpu.make_async_copy(ew_hbm.at[wid, pl.ds(0, PH)], ew_v,
                              sidx).wait()
        gather(0, rows_a, sga)

        def body(i, _):
            ca = 2 * i
            cb = 2 * i + 1
            gather_wait(rows_a, sga)
            scale(ca, rows_a)

            @pl.when(i > 0)
            def _():
                scatter_wait(rows_b, ssb)

            gather(cb, rows_b, sgb)
            scatter(ca, rows_a, ssa)
            gather_wait(rows_b, sgb)
            scale(cb, rows_b)
            scatter_wait(rows_a, ssa)

            @pl.when(i < PH // 2 - 1)
            def _():
                gather(ca + 2, rows_a, sga)

            scatter(cb, rows_b, ssb)
            return 0
        lax.fori_loop(0, PH // 2, body, 0)
        scatter_wait(rows_b, ssb)
        return 0
    lax.fori_loop(0, NPHASE, phase, 0)
    plsc.subcore_barrier()

    pltpu.sync_copy(acc_sh.at[pl.ds(sid * per, per)],
                    out_hbm.at[cid, pl.ds(sid * per, per)])


_BLK = 2000


def _tc_head(deg_ref, x_ref, w_ref, dis_ref, g_ref):
    deg = deg_ref[:, 0:1] + deg_ref[:, 1:2] + 1.0
    dis = lax.rsqrt(deg)
    dis_ref[...] = dis
    g_ref[...] = jnp.dot(x_ref[...], w_ref[...],
                         preferred_element_type=jnp.float32) * dis


def _tc_mid(acc_ref, g_ref, dis_ref, b_ref, w_ref, g2_ref):
    dis = dis_ref[...]
    t = dis * (acc_ref[0] + acc_ref[1] + g_ref[...]) + b_ref[...]
    h = jnp.maximum(t, 0.0)
    g2_ref[...] = jnp.dot(h, w_ref[...],
                          preferred_element_type=jnp.float32) * dis


def _tc_tail(acc_ref, g_ref, dis_ref, b_ref, out_ref):
    out_ref[...] = (dis_ref[...] * (acc_ref[0] + acc_ref[1] + g_ref[...])
                    + b_ref[...])


def _row_spec(block):
    return pl.BlockSpec((block, D), lambda i: (i, 0))


_tc_head_call = pl.pallas_call(
    _tc_head,
    grid=(N // _BLK,),
    in_specs=[
        pl.BlockSpec((_BLK, 2), lambda i: (i, 0)),
        _row_spec(_BLK),
        pl.BlockSpec((D, D), lambda i: (0, 0)),
    ],
    out_specs=[
        pl.BlockSpec((_BLK, 1), lambda i: (i, 0)),
        _row_spec(_BLK),
    ],
    out_shape=[
        jax.ShapeDtypeStruct((N, 1), jnp.float32),
        jax.ShapeDtypeStruct((N, D), jnp.float32),
    ],
)

_tc_mid_call = pl.pallas_call(
    _tc_mid,
    grid=(N // _BLK,),
    in_specs=[
        pl.BlockSpec((NC, _BLK, D), lambda i: (0, i, 0)),
        _row_spec(_BLK),
        pl.BlockSpec((_BLK, 1), lambda i: (i, 0)),
        pl.BlockSpec((1, D), lambda i: (0, 0)),
        pl.BlockSpec((D, D), lambda i: (0, 0)),
    ],
    out_specs=_row_spec(_BLK),
    out_shape=jax.ShapeDtypeStruct((N, D), jnp.float32),
)

_tc_tail_call = pl.pallas_call(
    _tc_tail,
    grid=(N // _BLK,),
    in_specs=[
        pl.BlockSpec((NC, _BLK, D), lambda i: (0, i, 0)),
        _row_spec(_BLK),
        pl.BlockSpec((_BLK, 1), lambda i: (i, 0)),
        pl.BlockSpec((1, D), lambda i: (0, 0)),
    ],
    out_specs=_row_spec(_BLK),
    out_shape=jax.ShapeDtypeStruct((N, D), jnp.float32),
)


def kernel(x, edge_index, edge_weight, W1, b1, W2, b2):
    src = edge_index[0].astype(jnp.int32)
    dst = edge_index[1].astype(jnp.int32)
    ew = edge_weight.astype(jnp.float32)
    pad = E_PAD - E
    src = jnp.concatenate([src, jnp.zeros((pad,), jnp.int32)])
    dst = jnp.concatenate([dst, jnp.zeros((pad,), jnp.int32)])
    ew = jnp.concatenate([ew, jnp.zeros((pad,), jnp.float32)])
    src3 = src.reshape(NW, NCHUNK, CH)
    dst3 = dst.reshape(NW, NCHUNK, CH)
    ew3 = ew.reshape(NW, NCHUNK, CH)

    deg_p = _deg_kernel(dst3, ew3)
    deg_t = jnp.transpose(deg_p.reshape(NC, N_PAD))
    dis, g1 = _tc_head_call(deg_t, x, W1)
    acc1 = _edge_kernel(g1, src3, dst3, ew3)
    g2 = _tc_mid_call(acc1, g1, dis, b1.reshape(1, D), W2)
    acc2 = _edge_kernel(g2, src3, dst3, ew3)
    out = _tc_tail_call(acc2, g2, dis, b2.reshape(1, D))
    return out
